# baseline (speedup 1.0000x reference)
import jax
import jax.numpy as jnp
from jax import lax
from jax.experimental import pallas as pl
from jax.experimental.pallas import tpu as pltpu, tpu_sc as plsc

_NC = 2
_NS = 16


def _tc_passthrough(x_ref, y_ref):
    pass


def kernel(x, demog_label, att_channel):
    B, C, H, W = x.shape
    nd = att_channel.shape[0]
    att2 = att_channel.reshape(nd, C)

    n_active = 16
    b_per_w = B // n_active

    mesh = plsc.VectorSubcoreMesh(core_axis_name="c", subcore_axis_name="s")

    def _sc_body(att_hbm, lab_hbm, att_out_hbm, idx_v, rows_v, att_v, sem):
        wid = lax.axis_index("s") * _NC + lax.axis_index("c")

        @pl.when(wid < n_active)
        def _gather():
            base = wid * b_per_w
            pltpu.sync_copy(lab_hbm.at[pl.ds(base, b_per_w)], idx_v)
            pltpu.async_copy(att_hbm.at[idx_v], rows_v, sem).wait()

        @pl.when(wid == n_active)
        def _att_copy():
            pltpu.sync_copy(att_hbm, att_v)
            pltpu.sync_copy(att_v, att_out_hbm)

    att_out = pl.kernel(
        _sc_body,
        out_type=jax.ShapeDtypeStruct((nd, C), jnp.float32),
        mesh=mesh,
        scratch_types=[
            pltpu.VMEM((b_per_w,), jnp.int32),
            pltpu.VMEM((b_per_w, C), jnp.float32),
            pltpu.VMEM((nd, C), jnp.float32),
            pltpu.SemaphoreType.DMA,
        ],
        name="att_row_gather_sc",
    )(att2, demog_label)

    y = pl.pallas_call(
        _tc_passthrough,
        in_specs=[pl.BlockSpec(memory_space=pl.ANY)],
        out_specs=pl.BlockSpec(memory_space=pl.ANY),
        out_shape=jax.ShapeDtypeStruct((B, C, H, W), jnp.float32),
        input_output_aliases={0: 0},
    )(x)

    return (y, att_out.reshape(att_channel.shape))


# speedup vs baseline: 7.5459x; 7.5459x over previous
import jax
import jax.numpy as jnp
from jax import lax
from jax.experimental import pallas as pl
from jax.experimental.pallas import tpu as pltpu, tpu_sc as plsc

_NC = 2
_NS = 16


def kernel(x, demog_label, att_channel):
    B, C, H, W = x.shape
    nd = att_channel.shape[0]
    att2 = att_channel.reshape(nd, C)

    n_active = 16
    b_per_w = B // n_active

    mesh = plsc.VectorSubcoreMesh(core_axis_name="c", subcore_axis_name="s")

    def _sc_body(att_hbm, lab_hbm, att_out_hbm, idx_v, rows_v, att_v, sem):
        wid = lax.axis_index("s") * _NC + lax.axis_index("c")

        @pl.when(wid < n_active)
        def _gather():
            base = wid * b_per_w
            pltpu.sync_copy(lab_hbm.at[pl.ds(base, b_per_w)], idx_v)
            pltpu.async_copy(att_hbm.at[idx_v], rows_v, sem).wait()

        @pl.when(wid == n_active)
        def _att_copy():
            pltpu.sync_copy(att_hbm, att_v)
            pltpu.sync_copy(att_v, att_out_hbm)

    att_out = pl.kernel(
        _sc_body,
        out_type=jax.ShapeDtypeStruct((nd, C), jnp.float32),
        mesh=mesh,
        scratch_types=[
            pltpu.VMEM((b_per_w,), jnp.int32),
            pltpu.VMEM((b_per_w, C), jnp.float32),
            pltpu.VMEM((nd, C), jnp.float32),
            pltpu.SemaphoreType.DMA,
        ],
        name="att_row_gather_sc",
    )(att2, demog_label)

    y = jnp.copy(x)
    probe = lax.squeeze(lax.slice(y, (0, 0, 0, 0), (1, 1, 1, 1)), (0, 1, 2, 3))
    att_dep, _ = lax.optimization_barrier((att_out, probe))

    return (y, att_dep.reshape(att_channel.shape))
